# Initial kernel scaffold; baseline (speedup 1.0000x reference)
import jax
import jax.numpy as jnp
from jax.experimental import pallas as pl


def kernel(x):
    raise NotImplementedError("write your pallas kernel here")


# speedup vs baseline: 1.8235x; 1.8235x over previous
import functools

import jax
import jax.numpy as jnp
from jax import lax
from jax.experimental import pallas as pl
from jax.experimental.pallas import tpu as pltpu
from jax.experimental.pallas import tpu_sc as plsc

IMG_W = 512
IMG_H = 512
BATCH = 64
NOUT = 129540
NWORKERS = 32
PAIR_OUT = 509
REG_PAIRS = 8
REG_IN = 16 * IMG_W
REG_OUT = REG_PAIRS * PAIR_OUT
TAIL_PAIRS = 7
TAIL_IN = 15 * IMG_W
TAIL_OUT = 6 * PAIR_OUT + 254 + 4
NOUT_PAD = NOUT + 4
IN_BUF = REG_IN + 16
OUT_BUF = REG_OUT + 16


def _body(x_ref, out_ref, inb0, inb1, outb0, outb1, is0, is1, os0, os1):
  nc = 2
  wid = lax.axis_index("s") * nc + lax.axis_index("c")

  iota = lax.iota(jnp.int32, 16)
  two_iota = iota * 2
  mixed15 = two_iota + 480 + jnp.where(
      iota < 14, jnp.full((16,), 2, jnp.int32), jnp.full((16,), 5, jnp.int32))

  inbs = (inb0, inb1)
  outbs = (outb0, outb1)
  isems = (is0, is1)
  osems = (os0, os1)

  def compute(inb, outb, npairs):
    for p in range(npairs):
      pb = 1024 * p
      for j in range(32):
        if j == 15:
          idx = mixed15 + pb
        else:
          c = 2 if j < 15 else 5
          idx = two_iota + (pb + 32 * j + c)
        v = plsc.load_gather(inb, [idx])
        outb[pl.ds(PAIR_OUT * p + 16 * j, 16)] = v

  def run(npairs, in_len, out_len):
    in_off = IMG_W * (16 * wid + 1)
    out_off = REG_OUT * wid

    def fire_in(b, d):
      pltpu.make_async_copy(x_ref.at[b, pl.ds(in_off, in_len)],
                            inbs[d].at[pl.ds(0, in_len)], isems[d]).start()

    def wait_in(d):
      pltpu.make_async_copy(x_ref.at[0, pl.ds(0, in_len)],
                            inbs[d].at[pl.ds(0, in_len)], isems[d]).wait()

    def fire_out(b, d):
      pltpu.make_async_copy(outbs[d].at[pl.ds(0, out_len)],
                            out_ref.at[b, pl.ds(out_off, out_len)],
                            osems[d]).start()

    def wait_out(d):
      pltpu.make_async_copy(x_ref.at[0, pl.ds(0, out_len)],
                            outbs[d].at[pl.ds(0, out_len)], osems[d]).wait()

    fire_in(0, 0)
    fire_in(1, 1)

    def step(i, carry):
      for d in range(2):
        b = 2 * i + d
        wait_in(d)
        pl.when(i >= 1)(lambda: wait_out(d))
        compute(inbs[d], outbs[d], npairs)
        fire_out(b, d)
        pl.when(i <= (BATCH // 2 - 2))(lambda: fire_in(b + 2, d))
      return carry

    lax.fori_loop(0, BATCH // 2, step, 0)
    wait_out(0)
    wait_out(1)

  pl.when(wid < NWORKERS - 1)(lambda: run(REG_PAIRS, REG_IN, REG_OUT))
  pl.when(wid == NWORKERS - 1)(lambda: run(TAIL_PAIRS, TAIL_IN, TAIL_OUT))


@jax.jit
def _line_filter(xf):
  mesh = plsc.VectorSubcoreMesh(core_axis_name="c", subcore_axis_name="s")
  return pl.kernel(
      _body,
      out_type=jax.ShapeDtypeStruct((BATCH, NOUT_PAD), jnp.float32),
      mesh=mesh,
      compiler_params=pltpu.CompilerParams(
          use_tc_tiling_on_sc=False, needs_layout_passes=False),
      scratch_types=[
          pltpu.VMEM((IN_BUF,), jnp.float32),
          pltpu.VMEM((IN_BUF,), jnp.float32),
          pltpu.VMEM((OUT_BUF,), jnp.float32),
          pltpu.VMEM((OUT_BUF,), jnp.float32),
          pltpu.SemaphoreType.DMA,
          pltpu.SemaphoreType.DMA,
          pltpu.SemaphoreType.DMA,
          pltpu.SemaphoreType.DMA,
      ],
  )(xf)


def kernel(x):
  xf = x.reshape(BATCH, IMG_H * IMG_W)
  return _line_filter(xf)[:, :NOUT]


# speedup vs baseline: 1.8595x; 1.0197x over previous
import jax
import jax.numpy as jnp
from jax import lax
from jax.experimental import pallas as pl
from jax.experimental.pallas import tpu as pltpu
from jax.experimental.pallas import tpu_sc as plsc

IMG_W = 512
IMG_H = 512
BATCH = 64
NOUT = 129540
NWORKERS = 32
PAIR_OUT = 509
SPAN = 2 * NOUT
NT1 = 62
T1_IN = 16 * IMG_W
T1_OUT = 8 * PAIR_OUT
BRIDGE_OUT = 6 * PAIR_OUT + 254 + 4 * PAIR_OUT
TAIL_OUT = 2 * PAIR_OUT + 254
IN_BUF = 15 * IMG_W + 8 * IMG_W + 16
OUT_BUF = BRIDGE_OUT + 16


def _compute(inb, outb, npairs, in_base=0, out_base=0):
  iota = lax.iota(jnp.int32, 16)
  two_iota = iota * 2
  mixed15 = two_iota + 480 + jnp.where(
      iota < 14, jnp.full((16,), 2, jnp.int32), jnp.full((16,), 5, jnp.int32))
  for p in range(npairs):
    pb = in_base + 1024 * p
    for j in range(32):
      if j == 15:
        idx = mixed15 + pb
      else:
        c = 2 if j < 15 else 5
        idx = two_iota + (pb + 32 * j + c)
      v = plsc.load_gather(inb, [idx])
      outb[pl.ds(out_base + PAIR_OUT * p + 16 * j, 16)] = v


def _body(x_ref, out_ref, inb0, inb1, outb0, outb1, is0, is1, os0, os1):
  nc = 2
  wid = lax.axis_index("s") * nc + lax.axis_index("c")
  a_base = wid * (2 * IMG_H * IMG_W)
  span = wid * SPAN

  inbs = (inb0, inb1)
  outbs = (outb0, outb1)
  isems = (is0, is1)
  osems = (os0, os1)

  def in_off(t):
    return a_base + 8192 * t + jnp.where(t < 31, 512, 12800)

  def out_off(t):
    return span + 4072 * t + jnp.where(t < 31, 0, 5344)

  def fire_in(t, d):
    pltpu.make_async_copy(x_ref.at[pl.ds(in_off(t), T1_IN)],
                          inbs[d].at[pl.ds(0, T1_IN)], isems[d]).start()

  def wait_in(d):
    pltpu.make_async_copy(x_ref.at[pl.ds(0, T1_IN)],
                          inbs[d].at[pl.ds(0, T1_IN)], isems[d]).wait()

  def fire_out(t, d):
    pltpu.make_async_copy(outbs[d].at[pl.ds(0, T1_OUT)],
                          out_ref.at[pl.ds(out_off(t), T1_OUT)],
                          osems[d]).start()

  def wait_out(d):
    pltpu.make_async_copy(x_ref.at[pl.ds(0, T1_OUT)],
                          outbs[d].at[pl.ds(0, T1_OUT)], osems[d]).wait()

  fire_in(0, 0)
  fire_in(1, 1)

  def step(i, carry):
    for d in range(2):
      t = 2 * i + d
      wait_in(d)
      pl.when(i >= 1)(lambda: wait_out(d))
      _compute(inbs[d], outbs[d], 8)
      fire_out(t, d)
      pl.when(i <= 29)(lambda: fire_in(t + 2, d))
    return carry

  lax.fori_loop(0, NT1 // 2, step, 0)
  wait_out(0)
  wait_out(1)

  pltpu.sync_copy(x_ref.at[pl.ds(a_base + 497 * IMG_W, 15 * IMG_W)],
                  inb0.at[pl.ds(0, 15 * IMG_W)])
  pltpu.sync_copy(x_ref.at[pl.ds(a_base + (IMG_H + 1) * IMG_W, 8 * IMG_W)],
                  inb0.at[pl.ds(15 * IMG_W, 8 * IMG_W)])
  _compute(inb0, outb0, 7)
  _compute(inb0, outb0, 4, in_base=15 * IMG_W, out_base=3308)
  pltpu.sync_copy(outb0.at[pl.ds(0, BRIDGE_OUT)],
                  out_ref.at[pl.ds(span + 31 * T1_OUT, BRIDGE_OUT)])

  pltpu.sync_copy(x_ref.at[pl.ds(a_base + IMG_H * IMG_W + 505 * IMG_W,
                                 7 * IMG_W)],
                  inb1.at[pl.ds(0, 7 * IMG_W)])
  _compute(inb1, outb1, 3)
  pltpu.sync_copy(outb1.at[pl.ds(0, TAIL_OUT)],
                  out_ref.at[pl.ds(span + SPAN - TAIL_OUT, TAIL_OUT)])


@jax.jit
def _line_filter(xf):
  mesh = plsc.VectorSubcoreMesh(core_axis_name="c", subcore_axis_name="s")
  return pl.kernel(
      _body,
      out_type=jax.ShapeDtypeStruct((BATCH * NOUT,), jnp.float32),
      mesh=mesh,
      compiler_params=pltpu.CompilerParams(
          use_tc_tiling_on_sc=False, needs_layout_passes=False),
      scratch_types=[
          pltpu.VMEM((IN_BUF,), jnp.float32),
          pltpu.VMEM((IN_BUF,), jnp.float32),
          pltpu.VMEM((OUT_BUF,), jnp.float32),
          pltpu.VMEM((OUT_BUF,), jnp.float32),
          pltpu.SemaphoreType.DMA,
          pltpu.SemaphoreType.DMA,
          pltpu.SemaphoreType.DMA,
          pltpu.SemaphoreType.DMA,
      ],
  )(xf)


def kernel(x):
  xf = x.reshape(BATCH * IMG_H * IMG_W)
  return _line_filter(xf).reshape(BATCH, NOUT)
